# baseline (speedup 1.0000x reference)
import functools

import jax
import jax.numpy as jnp
from jax import lax
from jax.experimental import pallas as pl
from jax.experimental.pallas import tpu as pltpu
from jax.experimental.pallas import tpu_sc as plsc

N = 8388608
N_SPECIES = 119
TBL = 128
NW = 32
PER_W = N // NW
C = 16384
NCHUNK = PER_W // C


def _mesh():
    return plsc.VectorSubcoreMesh(core_axis_name="c", subcore_axis_name="s")


@functools.partial(
    pl.kernel,
    out_type=jax.ShapeDtypeStruct((N,), jnp.float32),
    mesh=_mesh(),
    scratch_types=[
        pltpu.VMEM((TBL,), jnp.float32),
        pltpu.VMEM((TBL,), jnp.float32),
        pltpu.VMEM((TBL,), jnp.int32),
        pltpu.VMEM((2 * C,), jnp.float32),
        pltpu.VMEM((2 * C,), jnp.int32),
        pltpu.VMEM((2 * C,), jnp.float32),
        pltpu.SemaphoreType.DMA,
        pltpu.SemaphoreType.DMA,
        pltpu.SemaphoreType.DMA,
        pltpu.SemaphoreType.DMA,
    ],
    compiler_params=pltpu.CompilerParams(needs_layout_passes=False),
)
def _scale_shift_sc(x_hbm, z_hbm, scale_hbm, shift_hbm, out_hbm,
                    s_raw, t_raw, tbl, x_r, z_r, o_r,
                    in_sem0, in_sem1, out_sem0, out_sem1):
    wid = lax.axis_index("s") * 2 + lax.axis_index("c")
    base = wid * PER_W
    in_sems = (in_sem0, in_sem1)
    out_sems = (out_sem0, out_sem1)

    def start_in(ci, sel, sem):
        off = base + ci * C
        pltpu.async_copy(x_hbm.at[pl.ds(off, C)],
                         x_r.at[pl.ds(sel * C, C)], sem)
        pltpu.async_copy(z_hbm.at[pl.ds(off, C)],
                         z_r.at[pl.ds(sel * C, C)], sem)

    def wait_in(sem):
        pltpu.make_async_copy(x_hbm.at[pl.ds(0, C)],
                              x_r.at[pl.ds(0, C)], sem).wait()
        pltpu.make_async_copy(z_hbm.at[pl.ds(0, C)],
                              z_r.at[pl.ds(0, C)], sem).wait()

    def start_out(ci, sel, sem):
        off = base + ci * C
        pltpu.async_copy(o_r.at[pl.ds(sel * C, C)],
                         out_hbm.at[pl.ds(off, C)], sem)

    def wait_out(sem):
        pltpu.make_async_copy(o_r.at[pl.ds(0, C)],
                              out_hbm.at[pl.ds(0, C)], sem).wait()

    start_in(0, 0, in_sem0)
    start_in(1, 1, in_sem1)

    pltpu.sync_copy(scale_hbm, s_raw.at[pl.ds(0, N_SPECIES)])
    pltpu.sync_copy(shift_hbm, t_raw.at[pl.ds(0, N_SPECIES)])
    hi = jnp.int32(-65536)
    for k in range(TBL // 16):
        si = plsc.bitcast(s_raw[pl.ds(k * 16, 16)], jnp.int32)
        ti = plsc.bitcast(t_raw[pl.ds(k * 16, 16)], jnp.int32)
        tbl[pl.ds(k * 16, 16)] = (si & hi) | lax.shift_right_logical(ti, 16)

    def chunk_body(ci, carry):
        sel = lax.rem(ci, 2)

        @pl.when(sel == 0)
        def _():
            wait_in(in_sem0)

        @pl.when(sel == 1)
        def _():
            wait_in(in_sem1)

        @pl.when(jnp.logical_and(ci >= 2, sel == 0))
        def _():
            wait_out(out_sem0)

        @pl.when(jnp.logical_and(ci >= 2, sel == 1))
        def _():
            wait_out(out_sem1)

        sbase = sel * C

        @plsc.parallel_loop(0, C, step=16, unroll=8)
        def _vec(i):
            z = z_r[pl.ds(sbase + i, 16)]
            xv = x_r[pl.ds(sbase + i, 16)]
            w = plsc.load_gather(tbl, [z])
            s = plsc.bitcast(w & jnp.int32(-65536), jnp.float32)
            t = plsc.bitcast(lax.shift_left(w, 16), jnp.float32)
            o_r[pl.ds(sbase + i, 16)] = s * xv + t

        @pl.when(sel == 0)
        def _():
            start_out(ci, 0, out_sem0)

        @pl.when(sel == 1)
        def _():
            start_out(ci, 1, out_sem1)

        @pl.when(jnp.logical_and(ci + 2 < NCHUNK, sel == 0))
        def _():
            start_in(ci + 2, 0, in_sem0)

        @pl.when(jnp.logical_and(ci + 2 < NCHUNK, sel == 1))
        def _():
            start_in(ci + 2, 1, in_sem1)

        return carry

    lax.fori_loop(0, NCHUNK, chunk_body, 0)
    wait_out(out_sem0)
    wait_out(out_sem1)


def kernel(x, Z, scale_param, shift_param):
    return _scale_shift_sc(x, Z.astype(jnp.int32),
                           scale_param.astype(jnp.float32),
                           shift_param.astype(jnp.float32))


# speedup vs baseline: 1.0021x; 1.0021x over previous
import functools

import jax
import jax.numpy as jnp
from jax import lax
from jax.experimental import pallas as pl
from jax.experimental.pallas import tpu as pltpu
from jax.experimental.pallas import tpu_sc as plsc

N = 8388608
N_SPECIES = 119
TBL = 128
NW = 32
PER_W = N // NW
C = 16384
NCHUNK = PER_W // C
NPAIR = NCHUNK // 2


def _mesh():
    return plsc.VectorSubcoreMesh(core_axis_name="c", subcore_axis_name="s")


@functools.partial(
    pl.kernel,
    out_type=jax.ShapeDtypeStruct((N,), jnp.float32),
    mesh=_mesh(),
    scratch_types=[
        pltpu.VMEM((TBL,), jnp.float32),
        pltpu.VMEM((TBL,), jnp.float32),
        pltpu.VMEM((TBL,), jnp.int32),
        pltpu.VMEM((C,), jnp.float32),
        pltpu.VMEM((C,), jnp.float32),
        pltpu.VMEM((C,), jnp.int32),
        pltpu.VMEM((C,), jnp.int32),
        pltpu.VMEM((C,), jnp.float32),
        pltpu.VMEM((C,), jnp.float32),
        pltpu.SemaphoreType.DMA,
        pltpu.SemaphoreType.DMA,
        pltpu.SemaphoreType.DMA,
        pltpu.SemaphoreType.DMA,
    ],
    compiler_params=pltpu.CompilerParams(needs_layout_passes=False),
)
def _scale_shift_sc(x_hbm, z_hbm, scale_hbm, shift_hbm, out_hbm,
                    s_raw, t_raw, tbl, x_b0, x_b1, z_b0, z_b1, o_b0, o_b1,
                    in_sem0, in_sem1, out_sem0, out_sem1):
    wid = lax.axis_index("s") * 2 + lax.axis_index("c")
    base = wid * PER_W
    bufs = ((x_b0, z_b0, o_b0, in_sem0, out_sem0),
            (x_b1, z_b1, o_b1, in_sem1, out_sem1))

    def start_in(off, b):
        x_b, z_b, _, in_sem, _ = bufs[b]
        pltpu.async_copy(x_hbm.at[pl.ds(off, C)], x_b, in_sem)
        pltpu.async_copy(z_hbm.at[pl.ds(off, C)], z_b, in_sem)

    def wait_in(b):
        x_b, z_b, _, in_sem, _ = bufs[b]
        pltpu.make_async_copy(x_hbm.at[pl.ds(0, C)], x_b, in_sem).wait()
        pltpu.make_async_copy(z_hbm.at[pl.ds(0, C)], z_b, in_sem).wait()

    def start_out(off, b):
        _, _, o_b, _, out_sem = bufs[b]
        pltpu.async_copy(o_b, out_hbm.at[pl.ds(off, C)], out_sem)

    def wait_out(b):
        _, _, o_b, _, out_sem = bufs[b]
        pltpu.make_async_copy(o_b, out_hbm.at[pl.ds(0, C)], out_sem).wait()

    start_in(base, 0)
    start_in(base + C, 1)

    pltpu.sync_copy(scale_hbm, s_raw.at[pl.ds(0, N_SPECIES)])
    pltpu.sync_copy(shift_hbm, t_raw.at[pl.ds(0, N_SPECIES)])
    hi = jnp.int32(-65536)
    for k in range(TBL // 16):
        si = plsc.bitcast(s_raw[pl.ds(k * 16, 16)], jnp.int32)
        ti = plsc.bitcast(t_raw[pl.ds(k * 16, 16)], jnp.int32)
        tbl[pl.ds(k * 16, 16)] = (si & hi) | lax.shift_right_logical(ti, 16)

    def compute(b):
        x_b, z_b, o_b, _, _ = bufs[b]

        @plsc.parallel_loop(0, C, step=16, unroll=8)
        def _vec(i):
            z = z_b[pl.ds(i, 16)]
            xv = x_b[pl.ds(i, 16)]
            w = plsc.load_gather(tbl, [z])
            s = plsc.bitcast(w & jnp.int32(-65536), jnp.float32)
            t = plsc.bitcast(lax.shift_left(w, 16), jnp.float32)
            o_b[pl.ds(i, 16)] = s * xv + t

    def pair_body(p, carry):
        off0 = base + (2 * p) * C
        for b in range(2):
            off = off0 + b * C
            wait_in(b)

            @pl.when(p > 0)
            def _():
                wait_out(b)

            compute(b)
            start_out(off, b)

            @pl.when(p < NPAIR - 1)
            def _():
                start_in(off + 2 * C, b)

        return carry

    lax.fori_loop(0, NPAIR, pair_body, 0)
    wait_out(0)
    wait_out(1)


def kernel(x, Z, scale_param, shift_param):
    return _scale_shift_sc(x, Z.astype(jnp.int32),
                           scale_param.astype(jnp.float32),
                           shift_param.astype(jnp.float32))
